# Initial kernel scaffold; baseline (speedup 1.0000x reference)
import jax
import jax.numpy as jnp
from jax.experimental import pallas as pl


def kernel(hidden_states, q_k_mask, k_q_mask, Wq, Wk, Wv, Wfc, bfc, gamma, beta):
    raise NotImplementedError("write your pallas kernel here")


# speedup vs baseline: 10.4424x; 10.4424x over previous
import dataclasses
import functools
import math

import jax
import jax.numpy as jnp
from jax import lax
from jax.experimental import pallas as pl
from jax.experimental.pallas import tpu as pltpu
from jax.experimental.pallas import tpu_sc as plsc

_B, _S, _H, _D, _M, _DIN = 2, 2048, 8, 32, 32, 7
_CP = 16
_LW = _DIN * _M
_NC, _NS = 2, 16
_NW = _NC * _NS
_NIDX = _B * _S * _M
_BPW = _NIDX // _NW
_SPW = _BPW // _M
_TW = _S * _DIN


def _sc_gather_t(table_flat, idx_flat):
    mesh = plsc.VectorSubcoreMesh(core_axis_name="c", subcore_axis_name="s")
    cp = pltpu.CompilerParams()
    if "needs_layout_passes" in pltpu.CompilerParams.__dataclass_fields__:
        cp = dataclasses.replace(cp, needs_layout_passes=False)

    @functools.partial(
        pl.kernel,
        mesh=mesh,
        compiler_params=cp,
        out_type=jax.ShapeDtypeStruct((_B * _S * _LW,), jnp.float32),
        scratch_types=[
            pltpu.VMEM((_BPW,), jnp.int32),
            pltpu.VMEM((_TW,), jnp.float32),
            pltpu.VMEM((_SPW * _LW,), jnp.float32),
        ],
    )
    def gather_kernel(table_hbm, idx_hbm, out_hbm, idx_v, tab_v, out_v):
        wid = lax.axis_index("s") * _NC + lax.axis_index("c")
        bat = wid // _NS
        iwin = wid % _NS
        pltpu.sync_copy(idx_hbm.at[pl.ds(iwin * _BPW, _BPW)], idx_v)
        pltpu.sync_copy(table_hbm.at[pl.ds(bat * _TW, _TW)], tab_v)

        @pl.loop(0, _SPW)
        def _(s):
            for j in range(_M // 16):
                a = idx_v[pl.ds(s * _M + j * 16, 16)] * _DIN
                for c in range(_DIN):
                    out_v[pl.ds(s * _LW + c * _M + j * 16, 16)] = (
                        plsc.load_gather(tab_v, [a + c]))

        pltpu.sync_copy(out_v, out_hbm.at[pl.ds(wid * _SPW * _LW, _SPW * _LW)])

    return gather_kernel(table_flat, idx_flat)


def _tc_body(h_ref, hgt_ref, wq_ref, wkt_ref, wv_ref, wfc_ref, par_ref, out_ref):
    f32 = jnp.float32
    h16 = h_ref[...]
    hgt = hgt_ref[...]
    gamma = par_ref[0:1, :]
    beta = par_ref[1:2, :]
    bfc = par_ref[2:3, :]

    lane = lax.broadcasted_iota(jnp.int32, (1, _CP), 1)
    lmask = (lane < _DIN).astype(f32)
    mu = jnp.sum(h16, axis=1, keepdims=True) * (1.0 / _DIN)
    xc = (h16 - mu) * lmask
    var = jnp.sum(xc * xc, axis=1, keepdims=True) * (1.0 / _DIN)
    qn = xc * lax.rsqrt(var + 1e-6) * gamma + beta

    wq = wq_ref[...] * (1.0 / math.sqrt(_D))
    wkt = wkt_ref[...]
    a_blocks = []
    for hh in range(_H):
        a_blocks.append(
            jax.lax.dot(
                wq[:, hh * _D:(hh + 1) * _D],
                wkt[hh * _D:(hh + 1) * _D, :],
                precision="highest",
                preferred_element_type=f32,
            )
        )
    a_cat = jnp.concatenate(a_blocks, axis=1)
    qh = jax.lax.dot(qn, a_cat, precision="highest",
                     preferred_element_type=f32)

    u_cols = []
    for hh in range(_H):
        sc = qh[:, hh * _CP:hh * _CP + 1] * hgt[:, 0:_M]
        for c in range(1, _DIN):
            sc = sc + (qh[:, hh * _CP + c:hh * _CP + c + 1]
                       * hgt[:, c * _M:(c + 1) * _M])
        mx = jnp.max(sc, axis=1, keepdims=True)
        e = jnp.exp(sc - mx)
        w = e / jnp.sum(e, axis=1, keepdims=True)
        for c in range(_DIN):
            u_cols.append(jnp.sum(w * hgt[:, c * _M:(c + 1) * _M],
                                  axis=1, keepdims=True))
    u = jnp.concatenate(u_cols, axis=1)

    wv = wv_ref[...]
    wfc = wfc_ref[...]
    g_blocks = []
    for hh in range(_H):
        g_h = jax.lax.dot(
            wv[:, hh * _D:(hh + 1) * _D],
            wfc[hh * _D:(hh + 1) * _D, :],
            precision="highest",
            preferred_element_type=f32,
        )
        g_blocks.append(g_h[0:_DIN, :])
    g_cat = jnp.concatenate(g_blocks, axis=0)

    ctx = jax.lax.dot(u, g_cat, precision="highest",
                      preferred_element_type=f32)
    out_ref[...] = ctx + bfc + h16


def kernel(hidden_states, q_k_mask, k_q_mask, Wq, Wk, Wv, Wfc, bfc, gamma, beta):
    del k_q_mask
    f32 = jnp.float32
    h = hidden_states.astype(f32).reshape(_B * _S, _DIN)
    h_pad = jnp.pad(h, ((0, 0), (0, _CP - _DIN)))
    idx_flat = q_k_mask.astype(jnp.int32).reshape(_S * _M)

    hgt_all = _sc_gather_t(h.reshape(_B * _S * _DIN), idx_flat)
    hgt2 = hgt_all.reshape(_B * _S, _LW)

    wq_p = jnp.pad(Wq.astype(f32), ((0, _CP - _DIN), (0, 0)))
    wkt_p = jnp.pad(Wk.astype(f32).T, ((0, 0), (0, _CP - _DIN)))
    wv_p = jnp.pad(Wv.astype(f32), ((0, _CP - _DIN), (0, 0)))
    wfc_p = jnp.pad(Wfc.astype(f32), ((0, 0), (0, _CP - _DIN)))
    par = jnp.zeros((8, _CP), f32)
    par = par.at[0, :_DIN].set(gamma.astype(f32))
    par = par.at[1, :_DIN].set(beta.astype(f32))
    par = par.at[2, :_DIN].set(bfc.astype(f32))

    out = pl.pallas_call(
        _tc_body,
        grid=(_B,),
        in_specs=[
            pl.BlockSpec((_S, _CP), lambda b: (b, 0)),
            pl.BlockSpec((_S, _LW), lambda b: (b, 0)),
            pl.BlockSpec((_CP, _H * _D), lambda b: (0, 0)),
            pl.BlockSpec((_H * _D, _CP), lambda b: (0, 0)),
            pl.BlockSpec((_CP, _H * _D), lambda b: (0, 0)),
            pl.BlockSpec((_H * _D, _CP), lambda b: (0, 0)),
            pl.BlockSpec((8, _CP), lambda b: (0, 0)),
        ],
        out_specs=pl.BlockSpec((_S, _CP), lambda b: (b, 0)),
        out_shape=jax.ShapeDtypeStruct((_B * _S, _CP), f32),
    )(h_pad, hgt2, wq_p, wkt_p, wv_p, wfc_p, par)

    return out.reshape(_B, _S, _CP)[:, :, :_DIN]


# speedup vs baseline: 22.8224x; 2.1855x over previous
import dataclasses
import functools
import math

import jax
import jax.numpy as jnp
from jax import lax
from jax.experimental import pallas as pl
from jax.experimental.pallas import tpu as pltpu
from jax.experimental.pallas import tpu_sc as plsc

_B, _S, _H, _D, _M, _DIN = 2, 2048, 8, 32, 32, 7
_CP = 16
_LW = _DIN * _M
_NC, _NS = 2, 16
_NW = _NC * _NS
_NIDX = _B * _S * _M
_BPW = _NIDX // _NW
_SPW = _BPW // _M
_TW = _S * _DIN


def _sc_gather_t(table_flat, idx_flat):
    mesh = plsc.VectorSubcoreMesh(core_axis_name="c", subcore_axis_name="s")
    cp = pltpu.CompilerParams()
    if "needs_layout_passes" in pltpu.CompilerParams.__dataclass_fields__:
        cp = dataclasses.replace(cp, needs_layout_passes=False)

    @functools.partial(
        pl.kernel,
        mesh=mesh,
        compiler_params=cp,
        out_type=jax.ShapeDtypeStruct((_B * _S * _LW,), jnp.float32),
        scratch_types=[
            pltpu.VMEM((_BPW,), jnp.int32),
            pltpu.VMEM((_TW,), jnp.float32),
            pltpu.VMEM((_SPW * _LW,), jnp.float32),
        ],
    )
    def gather_kernel(table_hbm, idx_hbm, out_hbm, idx_v, tab_v, out_v):
        wid = lax.axis_index("s") * _NC + lax.axis_index("c")
        bat = wid // _NS
        iwin = wid % _NS
        pltpu.sync_copy(idx_hbm.at[pl.ds(iwin * _BPW, _BPW)], idx_v)
        pltpu.sync_copy(table_hbm.at[pl.ds(bat * _TW, _TW)], tab_v)

        @pl.loop(0, _SPW)
        def _(s):
            for j in range(_M // 16):
                a = idx_v[pl.ds(s * _M + j * 16, 16)] * _DIN
                for c in range(_DIN):
                    out_v[pl.ds(s * _LW + c * _M + j * 16, 16)] = (
                        plsc.load_gather(tab_v, [a + c]))

        pltpu.sync_copy(out_v, out_hbm.at[pl.ds(wid * _SPW * _LW, _SPW * _LW)])

    return gather_kernel(table_flat, idx_flat)


def _tc_body(h_ref, hgt_ref, wq_ref, wkt_ref, wv_ref, wfc_ref, par_ref, out_ref):
    f32 = jnp.float32
    h16 = h_ref[...]
    hgt = hgt_ref[...]
    gamma = par_ref[0:1, :]
    beta = par_ref[1:2, :]
    bfc = par_ref[2:3, :]

    lane = lax.broadcasted_iota(jnp.int32, (1, _CP), 1)
    lmask = (lane < _DIN).astype(f32)
    mu = jnp.sum(h16, axis=1, keepdims=True) * (1.0 / _DIN)
    xc = (h16 - mu) * lmask
    var = jnp.sum(xc * xc, axis=1, keepdims=True) * (1.0 / _DIN)
    qn = xc * lax.rsqrt(var + 1e-6) * gamma + beta

    bf16 = jnp.bfloat16

    wq = wq_ref[...] * (1.0 / math.sqrt(_D))
    wkt = wkt_ref[...]
    a_blocks = []
    for hh in range(_H):
        a_blocks.append(
            jax.lax.dot(
                wq[:, hh * _D:(hh + 1) * _D],
                wkt[hh * _D:(hh + 1) * _D, :],
                precision="highest",
                preferred_element_type=f32,
            )
        )
    a_cat2 = jnp.concatenate(
        [jnp.concatenate([a_blocks[hh][:, c:c + 1] for hh in range(_H)],
                         axis=1) for c in range(_DIN)], axis=1)
    qhc = jax.lax.dot(qn, a_cat2, precision="highest",
                      preferred_element_type=f32)

    wv = wv_ref[...]
    wfc = wfc_ref[...]
    g_blocks = []
    for hh in range(_H):
        g_h = jax.lax.dot(
            wv[:, hh * _D:(hh + 1) * _D],
            wfc[hh * _D:(hh + 1) * _D, :],
            precision="highest",
            preferred_element_type=f32,
        )
        g_blocks.append(g_h[0:_DIN, :])
    g_cat = jnp.concatenate(g_blocks, axis=0)

    hm = _H * _M
    rowh = lax.broadcasted_iota(jnp.int32, (_H, hm), 0)
    colh = lax.broadcasted_iota(jnp.int32, (_H, hm), 1)
    eh_f = (rowh == colh // _M).astype(f32)
    rowm = lax.broadcasted_iota(jnp.int32, (_M, hm), 0)
    colm = lax.broadcasted_iota(jnp.int32, (_M, hm), 1)
    et_bf = (rowm == colm % _M).astype(bf16)
    ri = lax.broadcasted_iota(jnp.int32, (hm, hm), 0)
    ci = lax.broadcasted_iota(jnp.int32, (hm, hm), 1)
    tones_bf = (ri // _M == ci // _M).astype(bf16)

    hgt_bf = hgt.astype(bf16)
    hbs = []
    sc = None
    for c in range(_DIN):
        qb = jax.lax.dot(qhc[:, c * _H:(c + 1) * _H], eh_f,
                         precision="highest", preferred_element_type=f32)
        hb = jax.lax.dot(hgt_bf[:, c * _M:(c + 1) * _M], et_bf,
                         preferred_element_type=f32)
        hbs.append(hb.astype(bf16))
        t = qb * hb
        sc = t if sc is None else sc + t

    mx = jnp.max(sc, axis=1, keepdims=True)
    e = jnp.exp(sc - mx)
    gsum = jax.lax.dot(e.astype(bf16), tones_bf,
                       preferred_element_type=f32)
    w_bf = (e / gsum).astype(bf16)

    ctx = None
    for c in range(_DIN):
        tgg_rows = []
        for hh in range(_H):
            tgg_rows.append(jnp.broadcast_to(g_blocks[hh][c:c + 1, :],
                                             (_M, _CP)))
        tgg_c = jnp.concatenate(tgg_rows, axis=0).astype(bf16)
        p = w_bf * hbs[c]
        t = jax.lax.dot(p, tgg_c, preferred_element_type=f32)
        ctx = t if ctx is None else ctx + t

    out_ref[...] = ctx + bfc + h16


def kernel(hidden_states, q_k_mask, k_q_mask, Wq, Wk, Wv, Wfc, bfc, gamma, beta):
    del k_q_mask
    f32 = jnp.float32
    h = hidden_states.astype(f32).reshape(_B * _S, _DIN)
    h_pad = jnp.pad(h, ((0, 0), (0, _CP - _DIN)))
    idx_flat = q_k_mask.astype(jnp.int32).reshape(_S * _M)

    hgt_all = _sc_gather_t(h.reshape(_B * _S * _DIN), idx_flat)
    hgt2 = hgt_all.reshape(_B * _S, _LW)

    wq_p = jnp.pad(Wq.astype(f32), ((0, _CP - _DIN), (0, 0)))
    wkt_p = jnp.pad(Wk.astype(f32).T, ((0, 0), (0, _CP - _DIN)))
    wv_p = jnp.pad(Wv.astype(f32), ((0, _CP - _DIN), (0, 0)))
    wfc_p = jnp.pad(Wfc.astype(f32), ((0, 0), (0, _CP - _DIN)))
    par = jnp.zeros((8, _CP), f32)
    par = par.at[0, :_DIN].set(gamma.astype(f32))
    par = par.at[1, :_DIN].set(beta.astype(f32))
    par = par.at[2, :_DIN].set(bfc.astype(f32))

    out = pl.pallas_call(
        _tc_body,
        grid=(_B,),
        in_specs=[
            pl.BlockSpec((_S, _CP), lambda b: (b, 0)),
            pl.BlockSpec((_S, _LW), lambda b: (b, 0)),
            pl.BlockSpec((_CP, _H * _D), lambda b: (0, 0)),
            pl.BlockSpec((_H * _D, _CP), lambda b: (0, 0)),
            pl.BlockSpec((_CP, _H * _D), lambda b: (0, 0)),
            pl.BlockSpec((_H * _D, _CP), lambda b: (0, 0)),
            pl.BlockSpec((8, _CP), lambda b: (0, 0)),
        ],
        out_specs=pl.BlockSpec((_S, _CP), lambda b: (b, 0)),
        out_shape=jax.ShapeDtypeStruct((_B * _S, _CP), f32),
    )(h_pad, hgt2, wq_p, wkt_p, wv_p, wfc_p, par)

    return out.reshape(_B, _S, _CP)[:, :, :_DIN]


# speedup vs baseline: 25.1701x; 1.1029x over previous
import dataclasses
import functools
import math

import jax
import jax.numpy as jnp
from jax import lax
from jax.experimental import pallas as pl
from jax.experimental.pallas import tpu as pltpu
from jax.experimental.pallas import tpu_sc as plsc

_B, _S, _H, _D, _M, _DIN = 2, 2048, 8, 32, 32, 7
_LW = _DIN * _M
_HM = _H * _M
_NC, _NS = 2, 16
_NW = _NC * _NS
_NIDX = _B * _S * _M
_BPW = _NIDX // _NW
_SPW = _BPW // _M
_TW = _S * _DIN


def _sc_gather_t(table_flat, idx_flat):
    mesh = plsc.VectorSubcoreMesh(core_axis_name="c", subcore_axis_name="s")
    cp = pltpu.CompilerParams()
    if "needs_layout_passes" in pltpu.CompilerParams.__dataclass_fields__:
        cp = dataclasses.replace(cp, needs_layout_passes=False)

    @functools.partial(
        pl.kernel,
        mesh=mesh,
        compiler_params=cp,
        out_type=jax.ShapeDtypeStruct((_B * _S * _LW,), jnp.float32),
        scratch_types=[
            pltpu.VMEM((_BPW,), jnp.int32),
            pltpu.VMEM((_TW,), jnp.float32),
            pltpu.VMEM((_SPW * _LW,), jnp.float32),
        ],
    )
    def gather_kernel(table_hbm, idx_hbm, out_hbm, idx_v, tab_v, out_v):
        wid = lax.axis_index("s") * _NC + lax.axis_index("c")
        bat = wid // _NS
        iwin = wid % _NS
        pltpu.sync_copy(idx_hbm.at[pl.ds(iwin * _BPW, _BPW)], idx_v)
        pltpu.sync_copy(table_hbm.at[pl.ds(bat * _TW, _TW)], tab_v)

        @plsc.parallel_loop(0, _SPW, unroll=4)
        def _(s):
            for j in range(_M // 16):
                a = idx_v[pl.ds(s * _M + j * 16, 16)] * _DIN
                for c in range(_DIN):
                    out_v[pl.ds(s * _LW + c * _M + j * 16, 16)] = (
                        plsc.load_gather(tab_v, [a + c]))

        pltpu.sync_copy(out_v, out_hbm.at[pl.ds(wid * _SPW * _LW, _SPW * _LW)])

    return gather_kernel(table_flat, idx_flat)


def _tc_body(h_ref, hgt_ref, wq_ref, wk_ref, wv_ref, wfc_ref, par_ref, out_ref):
    f32 = jnp.float32
    bf16 = jnp.bfloat16
    h7 = h_ref[...]
    hgt = hgt_ref[...]
    gamma = par_ref[0:1, :]
    beta = par_ref[1:2, :]
    bfc = par_ref[2:3, :]

    mu = jnp.sum(h7, axis=1, keepdims=True) * (1.0 / _DIN)
    xc = h7 - mu
    var = jnp.sum(xc * xc, axis=1, keepdims=True) * (1.0 / _DIN)
    qn = xc * lax.rsqrt(var + 1e-6) * gamma + beta

    wq = wq_ref[...] * (1.0 / math.sqrt(_D))
    wk = wk_ref[...]
    tdn = (((1,), (1,)), ((), ()))
    a_blocks = []
    for hh in range(_H):
        a_blocks.append(
            lax.dot_general(
                wq[:, hh * _D:(hh + 1) * _D],
                wk[:, hh * _D:(hh + 1) * _D],
                tdn,
                precision="highest",
                preferred_element_type=f32,
            )
        )
    a_cat2 = jnp.concatenate(
        [jnp.concatenate([a_blocks[hh][:, c:c + 1] for hh in range(_H)],
                         axis=1) for c in range(_DIN)], axis=1)
    qhc = jax.lax.dot(qn, a_cat2, precision="highest",
                      preferred_element_type=f32)

    wv = wv_ref[...]
    wfc = wfc_ref[...]
    g_blocks = []
    for hh in range(_H):
        g_blocks.append(
            jax.lax.dot(
                wv[:, hh * _D:(hh + 1) * _D],
                wfc[hh * _D:(hh + 1) * _D, :],
                precision="highest",
                preferred_element_type=f32,
            )
        )

    rowh = lax.broadcasted_iota(jnp.int32, (_H, _HM), 0)
    colh = lax.broadcasted_iota(jnp.int32, (_H, _HM), 1)
    eh_f = (rowh == colh // _M).astype(f32)
    rowm = lax.broadcasted_iota(jnp.int32, (_M, _HM), 0)
    colm = lax.broadcasted_iota(jnp.int32, (_M, _HM), 1)
    et_bf = (rowm == colm % _M).astype(bf16)
    ri = lax.broadcasted_iota(jnp.int32, (_HM, _HM), 0)
    ci = lax.broadcasted_iota(jnp.int32, (_HM, _HM), 1)
    tones_bf = (ri // _M == ci // _M).astype(bf16)

    hgt_bf = hgt.astype(bf16)
    hbs = []
    sc = None
    for c in range(_DIN):
        qb = jax.lax.dot(qhc[:, c * _H:(c + 1) * _H], eh_f,
                         precision="highest", preferred_element_type=f32)
        hb = jax.lax.dot(hgt_bf[:, c * _M:(c + 1) * _M], et_bf,
                         preferred_element_type=f32)
        hbs.append(hb.astype(bf16))
        t = qb * hb
        sc = t if sc is None else sc + t

    mx = jnp.max(sc, axis=1, keepdims=True)
    e = jnp.exp(sc - mx)
    gsum = jax.lax.dot(e.astype(bf16), tones_bf,
                       preferred_element_type=f32)
    w_bf = (e / gsum).astype(bf16)

    ctx = None
    for c in range(_DIN):
        tgg_rows = []
        for hh in range(_H):
            tgg_rows.append(jnp.broadcast_to(g_blocks[hh][c:c + 1, :],
                                             (_M, _DIN)))
        tgg_c = jnp.concatenate(tgg_rows, axis=0).astype(bf16)
        p = w_bf * hbs[c]
        t = jax.lax.dot(p, tgg_c, preferred_element_type=f32)
        ctx = t if ctx is None else ctx + t

    out_ref[...] = ctx + bfc + h7


def kernel(hidden_states, q_k_mask, k_q_mask, Wq, Wk, Wv, Wfc, bfc, gamma, beta):
    del k_q_mask
    f32 = jnp.float32
    h = hidden_states.reshape(_B * _S, _DIN)
    idx_flat = q_k_mask.astype(jnp.int32).reshape(_S * _M)

    hgt_all = _sc_gather_t(h.reshape(_B * _S * _DIN), idx_flat)
    hgt2 = hgt_all.reshape(_B * _S, _LW)

    par = jnp.stack([gamma, beta, bfc]).astype(f32)

    out = pl.pallas_call(
        _tc_body,
        grid=(_B,),
        in_specs=[
            pl.BlockSpec((_S, _DIN), lambda b: (b, 0)),
            pl.BlockSpec((_S, _LW), lambda b: (b, 0)),
            pl.BlockSpec((_DIN, _H * _D), lambda b: (0, 0)),
            pl.BlockSpec((_DIN, _H * _D), lambda b: (0, 0)),
            pl.BlockSpec((_DIN, _H * _D), lambda b: (0, 0)),
            pl.BlockSpec((_H * _D, _DIN), lambda b: (0, 0)),
            pl.BlockSpec((3, _DIN), lambda b: (0, 0)),
        ],
        out_specs=pl.BlockSpec((_S, _DIN), lambda b: (b, 0)),
        out_shape=jax.ShapeDtypeStruct((_B * _S, _DIN), f32),
    )(h, hgt2, Wq, Wk, Wv, Wfc, par)

    return out.reshape(_B, _S, _DIN)
